# baseline (speedup 1.0000x reference)
import functools

import jax
import jax.numpy as jnp
from jax import lax
from jax.experimental import pallas as pl
from jax.experimental.pallas import tpu as pltpu
from jax.experimental.pallas import tpu_sc as plsc


def _tc_expand_body(x_ref, o_ref):
    f32 = jnp.float32
    i0 = lax.broadcasted_iota(jnp.int32, (128, 2048), 0)
    i1 = lax.broadcasted_iota(jnp.int32, (128, 2048), 1)
    Q = (i0 == 8 * (i1 // 128) + (i1 % 128) // 16).astype(f32)
    x = x_ref[...].astype(f32)
    y = jnp.dot(x, Q, preferred_element_type=f32)
    o_ref[...] = y.astype(jnp.int32)


def _tc_expand(seq2d):
    R = seq2d.shape[0]
    return pl.pallas_call(
        _tc_expand_body,
        out_shape=jax.ShapeDtypeStruct((R, 2048), jnp.int32),
    )(seq2d)


def _make_sc_counter(n_total, num_cores, num_subcores):
    W = num_cores * num_subcores
    L = n_total // W
    mesh = plsc.VectorSubcoreMesh(core_axis_name="c", subcore_axis_name="s")

    H = L // 2

    @functools.partial(
        pl.kernel,
        mesh=mesh,
        out_type=jax.ShapeDtypeStruct((W * 256,), jnp.int32),
        scratch_types=[
            pltpu.VMEM((L * 16,), jnp.int32),
            pltpu.VMEM((256,), jnp.int32),
        ],
    )
    def sc_count(seq_hbm, out_hbm, seq_v, out_v):
        wid = lax.axis_index("s") * num_cores + lax.axis_index("c")
        pltpu.sync_copy(seq_hbm.at[pl.ds(wid * (L * 16), L * 16)], seq_v)

        iota = lax.iota(jnp.int32, 16)
        zero = jnp.zeros((16,), jnp.int32)
        one = jnp.ones((16,), jnp.int32)
        mcol = iota & 3
        mrow = iota >> 2

        def step(st, c):
            n1r, n2, n30, n31, n32, s2 = st
            s2 = s2 + n2
            n30 = n30 + jnp.where(c == 0, n2, zero)
            n31 = n31 + jnp.where(c == 1, n2, zero)
            n32 = n32 + jnp.where(c == 2, n2, zero)
            n2 = n2 + jnp.where(c == mcol, n1r, zero)
            n1r = n1r + jnp.where(c == mrow, one, zero)
            return (n1r, n2, n30, n31, n32, s2)

        def body(i, carry):
            stA, stB = carry[:6], carry[6:]
            base = i * 4 * 16
            for u in range(4):
                stA = step(stA, seq_v[pl.ds(base + u * 16, 16)])
                stB = step(stB, seq_v[pl.ds(base + u * 16 + H * 16, 16)])
            return stA + stB

        st = lax.fori_loop(0, H // 4, body, (zero,) * 12)

        for half in range(2):
            n1r, n2, n30, n31, n32, s2 = st[half * 6:half * 6 + 6]
            o = half * 128
            out_v[pl.ds(o + 0, 16)] = n1r
            out_v[pl.ds(o + 16, 16)] = n2
            out_v[pl.ds(o + 32, 16)] = n30
            out_v[pl.ds(o + 48, 16)] = n31
            out_v[pl.ds(o + 64, 16)] = n32
            out_v[pl.ds(o + 80, 16)] = s2 - n30 - n31 - n32
            out_v[pl.ds(o + 96, 16)] = zero
            out_v[pl.ds(o + 112, 16)] = zero
        pltpu.sync_copy(out_v, out_hbm.at[pl.ds(wid * 256, 256)])

    return sc_count, 2 * W


def _tc_finalize_body(W, n_total,
                      p_ref, h_ref, s_ref, gs_ref,
                      cs_ref, dw_ref, cp_ref, o_ref):
    f32 = jnp.float32
    dot = functools.partial(jnp.dot, preferred_element_type=f32,
                            precision=lax.Precision.HIGHEST)

    Pf = p_ref[...].astype(f32)
    i160 = lax.broadcasted_iota(jnp.int32, (16, 4), 0)
    i161 = lax.broadcasted_iota(jnp.int32, (16, 4), 1)
    SEL = (i160 == i161 * 4).astype(f32)
    N1b = dot(Pf[:, 0:16], SEL)
    N2b = Pf[:, 16:32]
    N3b = Pf[:, 32:96]

    iw0 = lax.broadcasted_iota(jnp.int32, (W, W), 0)
    iw1 = lax.broadcasted_iota(jnp.int32, (W, W), 1)
    Lstrict = (iw1 < iw0).astype(f32)

    N1pre = dot(Lstrict, N1b)

    i40 = lax.broadcasted_iota(jnp.int32, (4, 16), 0)
    i41 = lax.broadcasted_iota(jnp.int32, (4, 16), 1)
    E4 = (i41 // 4 == i40).astype(f32)
    F4 = (i41 % 4 == i40).astype(f32)
    N1pre16 = dot(N1pre, E4)
    N1b16 = dot(N1b, F4)
    S2b = N2b + N1pre16 * N1b16
    N2pre = dot(Lstrict, S2b)

    ia0 = lax.broadcasted_iota(jnp.int32, (4, 64), 0)
    ia1 = lax.broadcasted_iota(jnp.int32, (4, 64), 1)
    E16 = (ia1 // 16 == ia0).astype(f32)
    ig0 = lax.broadcasted_iota(jnp.int32, (16, 64), 0)
    ig1 = lax.broadcasted_iota(jnp.int32, (16, 64), 1)
    Etile = (ig1 % 16 == ig0).astype(f32)

    ones_w = jnp.ones((1, W), f32)
    A64 = dot(ones_w, N3b)
    Bflat = dot(ones_w, dot(N1b, E16) * dot(N2pre, Etile))
    ABflat = A64 + Bflat
    Cflat = dot(ones_w, dot(N1pre, E16) * dot(N2b, Etile))

    f0 = lax.broadcasted_iota(jnp.int32, (64, 64), 0)
    d1 = lax.broadcasted_iota(jnp.int32, (64, 64), 1)

    def cell_matrix(aa, bb, cc):
        sh = jnp.zeros((64, 64), jnp.int32)
        wv = jnp.ones((64, 64), f32)
        for k in range(4):
            sh = sh + jnp.where(aa == k, h_ref[0, k], 0)
            sh = sh + jnp.where(bb == k, h_ref[1, k], 0)
            sh = sh + jnp.where(cc == k, h_ref[2, k], 0)
            wv = wv * jnp.where(aa == k, s_ref[0, k], 1.0)
            wv = wv * jnp.where(bb == k, s_ref[1, k], 1.0)
            wv = wv * jnp.where(cc == k, s_ref[2, k], 1.0)
        return (sh % 64 == d1).astype(f32) * wv

    M_AB = cell_matrix((f0 // 4) % 4, f0 % 4, f0 // 16)
    M_C = cell_matrix(f0 // 16, (f0 // 4) % 4, f0 % 4)
    sk = dot(ABflat, M_AB) + dot(Cflat, M_C)

    N1tot = dot(ones_w, N1b)
    probs = N1tot / float(n_total)
    scaling = 0.9 + 0.2 * jnp.sum(probs * cs_ref[...])
    nrm = jnp.sqrt(jnp.sum(sk * sk))
    base = sk / (nrm + 1e-8)
    out = base * gs_ref[0, 0] * scaling * dw_ref[...]
    out = out + 0.1 * dot(probs, cp_ref[...])
    o_ref[...] = out


def _tc_finalize(W, n_total, p, h_i, s_f, gs, cs, dw, cp):
    smem = pl.BlockSpec(memory_space=pltpu.SMEM)
    vmem = pl.BlockSpec(memory_space=pltpu.VMEM)
    return pl.pallas_call(
        functools.partial(_tc_finalize_body, W, n_total),
        out_shape=jax.ShapeDtypeStruct((1, 64), jnp.float32),
        in_specs=[vmem, smem, smem, smem, vmem, vmem, vmem],
    )(p, h_i, s_f, gs, cs, dw, cp)


def kernel(sequence, h, s, global_scale, char_scales, dim_weights,
           char_perturbations):
    n_total = sequence.shape[0]
    info = plsc.get_sparse_core_info()
    sc_count, W = _make_sc_counter(n_total, info.num_cores, info.num_subcores)

    seq2d = sequence.astype(jnp.int32).reshape(n_total // 128, 128)
    seq_exp = _tc_expand(seq2d).reshape(n_total * 16)
    partials = sc_count(seq_exp).reshape(W, 128)

    out = _tc_finalize(
        W, n_total, partials,
        h.astype(jnp.int32), s.astype(jnp.float32),
        jnp.reshape(global_scale, (1, 1)).astype(jnp.float32),
        jnp.reshape(char_scales, (1, 4)).astype(jnp.float32),
        jnp.reshape(dim_weights, (1, 64)).astype(jnp.float32),
        char_perturbations.astype(jnp.float32),
    )
    return out.reshape(64)


# speedup vs baseline: 1.0882x; 1.0882x over previous
import functools

import jax
import jax.numpy as jnp
from jax import lax
from jax.experimental import pallas as pl
from jax.experimental.pallas import tpu as pltpu
from jax.experimental.pallas import tpu_sc as plsc


def _tc_expand_body(x_ref, o_ref):
    f32 = jnp.float32
    i0 = lax.broadcasted_iota(jnp.int32, (128, 2048), 0)
    i1 = lax.broadcasted_iota(jnp.int32, (128, 2048), 1)
    Q = (i0 == 8 * (i1 // 128) + (i1 % 128) // 16).astype(f32)
    x = x_ref[...].astype(f32)
    y = jnp.dot(x, Q, preferred_element_type=f32)
    o_ref[...] = y.astype(jnp.int32)


def _tc_expand(seq2d):
    R = seq2d.shape[0]
    return pl.pallas_call(
        _tc_expand_body,
        out_shape=jax.ShapeDtypeStruct((R, 2048), jnp.int32),
    )(seq2d)


def _make_sc_counter(n_total, num_cores, num_subcores):
    W = num_cores * num_subcores
    L = n_total // W
    mesh = plsc.VectorSubcoreMesh(core_axis_name="c", subcore_axis_name="s")

    H = L // 2

    @functools.partial(
        pl.kernel,
        mesh=mesh,
        out_type=jax.ShapeDtypeStruct((W * 256,), jnp.int32),
        scratch_types=[
            pltpu.VMEM((L * 16,), jnp.int32),
            pltpu.VMEM((256,), jnp.int32),
        ],
    )
    def sc_count(seq_hbm, out_hbm, seq_v, out_v):
        wid = lax.axis_index("s") * num_cores + lax.axis_index("c")
        pltpu.sync_copy(seq_hbm.at[pl.ds(wid * (L * 16), L * 16)], seq_v)

        iota = lax.iota(jnp.int32, 16)
        zero = jnp.zeros((16,), jnp.int32)
        one = jnp.ones((16,), jnp.int32)
        mcol = iota & 3
        mrow = iota >> 2

        def step(st, c):
            n1r, n2, n30, n31, n32, s2 = st
            s2 = s2 + n2
            n30 = n30 + jnp.where(c == 0, n2, zero)
            n31 = n31 + jnp.where(c == 1, n2, zero)
            n32 = n32 + jnp.where(c == 2, n2, zero)
            n2 = n2 + jnp.where(c == mcol, n1r, zero)
            n1r = n1r + jnp.where(c == mrow, one, zero)
            return (n1r, n2, n30, n31, n32, s2)

        def body(i, carry):
            stA, stB = carry[:6], carry[6:]
            base = i * 16 * 16
            for u in range(16):
                stA = step(stA, seq_v[pl.ds(base + u * 16, 16)])
                stB = step(stB, seq_v[pl.ds(base + u * 16 + H * 16, 16)])
            return stA + stB

        st = lax.fori_loop(0, H // 16, body, (zero,) * 12)

        for half in range(2):
            n1r, n2, n30, n31, n32, s2 = st[half * 6:half * 6 + 6]
            o = half * 128
            out_v[pl.ds(o + 0, 16)] = n1r
            out_v[pl.ds(o + 16, 16)] = n2
            out_v[pl.ds(o + 32, 16)] = n30
            out_v[pl.ds(o + 48, 16)] = n31
            out_v[pl.ds(o + 64, 16)] = n32
            out_v[pl.ds(o + 80, 16)] = s2 - n30 - n31 - n32
            out_v[pl.ds(o + 96, 16)] = zero
            out_v[pl.ds(o + 112, 16)] = zero
        pltpu.sync_copy(out_v, out_hbm.at[pl.ds(wid * 256, 256)])

    return sc_count, 2 * W


def _tc_finalize_body(W, n_total,
                      p_ref, h_ref, s_ref, gs_ref,
                      cs_ref, dw_ref, cp_ref, o_ref):
    f32 = jnp.float32
    dot = functools.partial(jnp.dot, preferred_element_type=f32)

    Pf = p_ref[...].astype(f32)
    i160 = lax.broadcasted_iota(jnp.int32, (16, 4), 0)
    i161 = lax.broadcasted_iota(jnp.int32, (16, 4), 1)
    SEL = (i160 == i161 * 4).astype(f32)
    N1b = dot(Pf[:, 0:16], SEL)
    N2b = Pf[:, 16:32]
    N3b = Pf[:, 32:96]

    iw0 = lax.broadcasted_iota(jnp.int32, (W, W), 0)
    iw1 = lax.broadcasted_iota(jnp.int32, (W, W), 1)
    Lstrict = (iw1 < iw0).astype(f32)

    N1pre = dot(Lstrict, N1b)

    i40 = lax.broadcasted_iota(jnp.int32, (4, 16), 0)
    i41 = lax.broadcasted_iota(jnp.int32, (4, 16), 1)
    E4 = (i41 // 4 == i40).astype(f32)
    F4 = (i41 % 4 == i40).astype(f32)
    N1pre16 = dot(N1pre, E4)
    N1b16 = dot(N1b, F4)
    S2b = N2b + N1pre16 * N1b16
    N2pre = dot(Lstrict, S2b)

    ia0 = lax.broadcasted_iota(jnp.int32, (4, 64), 0)
    ia1 = lax.broadcasted_iota(jnp.int32, (4, 64), 1)
    E16 = (ia1 // 16 == ia0).astype(f32)
    ig0 = lax.broadcasted_iota(jnp.int32, (16, 64), 0)
    ig1 = lax.broadcasted_iota(jnp.int32, (16, 64), 1)
    Etile = (ig1 % 16 == ig0).astype(f32)

    ones_w = jnp.ones((1, W), f32)
    A64 = dot(ones_w, N3b)
    Bflat = dot(ones_w, dot(N1b, E16) * dot(N2pre, Etile))
    ABflat = A64 + Bflat
    Cflat = dot(ones_w, dot(N1pre, E16) * dot(N2b, Etile))

    f0 = lax.broadcasted_iota(jnp.int32, (64, 64), 0)
    d1 = lax.broadcasted_iota(jnp.int32, (64, 64), 1)

    def cell_matrix(aa, bb, cc):
        sh = jnp.zeros((64, 64), jnp.int32)
        wv = jnp.ones((64, 64), f32)
        for k in range(4):
            sh = sh + jnp.where(aa == k, h_ref[0, k], 0)
            sh = sh + jnp.where(bb == k, h_ref[1, k], 0)
            sh = sh + jnp.where(cc == k, h_ref[2, k], 0)
            wv = wv * jnp.where(aa == k, s_ref[0, k], 1.0)
            wv = wv * jnp.where(bb == k, s_ref[1, k], 1.0)
            wv = wv * jnp.where(cc == k, s_ref[2, k], 1.0)
        return (sh % 64 == d1).astype(f32) * wv

    M_AB = cell_matrix((f0 // 4) % 4, f0 % 4, f0 // 16)
    M_C = cell_matrix(f0 // 16, (f0 // 4) % 4, f0 % 4)
    sk = dot(ABflat, M_AB) + dot(Cflat, M_C)

    N1tot = dot(ones_w, N1b)
    probs = N1tot / float(n_total)
    scaling = 0.9 + 0.2 * jnp.sum(probs * cs_ref[...])
    nrm = jnp.sqrt(jnp.sum(sk * sk))
    base = sk / (nrm + 1e-8)
    out = base * gs_ref[0, 0] * scaling * dw_ref[...]
    out = out + 0.1 * dot(probs, cp_ref[...])
    o_ref[...] = out


def _tc_finalize(W, n_total, p, h_i, s_f, gs, cs, dw, cp):
    smem = pl.BlockSpec(memory_space=pltpu.SMEM)
    vmem = pl.BlockSpec(memory_space=pltpu.VMEM)
    return pl.pallas_call(
        functools.partial(_tc_finalize_body, W, n_total),
        out_shape=jax.ShapeDtypeStruct((1, 64), jnp.float32),
        in_specs=[vmem, smem, smem, smem, vmem, vmem, vmem],
    )(p, h_i, s_f, gs, cs, dw, cp)


def kernel(sequence, h, s, global_scale, char_scales, dim_weights,
           char_perturbations):
    n_total = sequence.shape[0]
    info = plsc.get_sparse_core_info()
    sc_count, W = _make_sc_counter(n_total, info.num_cores, info.num_subcores)

    seq2d = sequence.astype(jnp.int32).reshape(n_total // 128, 128)
    seq_exp = _tc_expand(seq2d).reshape(n_total * 16)
    partials = sc_count(seq_exp).reshape(W, 128)

    out = _tc_finalize(
        W, n_total, partials,
        h.astype(jnp.int32), s.astype(jnp.float32),
        jnp.reshape(global_scale, (1, 1)).astype(jnp.float32),
        jnp.reshape(char_scales, (1, 4)).astype(jnp.float32),
        jnp.reshape(dim_weights, (1, 64)).astype(jnp.float32),
        char_perturbations.astype(jnp.float32),
    )
    return out.reshape(64)


# speedup vs baseline: 1.1822x; 1.0864x over previous
import functools

import jax
import jax.numpy as jnp
from jax import lax
from jax.experimental import pallas as pl
from jax.experimental.pallas import tpu as pltpu
from jax.experimental.pallas import tpu_sc as plsc


def _tc_expand_body(x_ref, o_ref):
    f32 = jnp.float32
    i0 = lax.broadcasted_iota(jnp.int32, (128, 2048), 0)
    i1 = lax.broadcasted_iota(jnp.int32, (128, 2048), 1)
    Q = (i0 == 8 * (i1 // 128) + (i1 % 128) // 16).astype(f32)
    x = x_ref[...].astype(f32)
    y = jnp.dot(x, Q, preferred_element_type=f32)
    o_ref[...] = y.astype(jnp.int32).reshape(o_ref.shape)


def _tc_expand(seq2d):
    R = seq2d.shape[0]
    return pl.pallas_call(
        _tc_expand_body,
        out_shape=jax.ShapeDtypeStruct((R * 16, 128), jnp.int32),
    )(seq2d)


def _make_sc_counter(n_total, num_cores, num_subcores):
    W = num_cores * num_subcores
    L = n_total // W
    mesh = plsc.VectorSubcoreMesh(core_axis_name="c", subcore_axis_name="s")

    H = L // 2

    @functools.partial(
        pl.kernel,
        mesh=mesh,
        out_type=jax.ShapeDtypeStruct((W * 256,), jnp.int32),
        scratch_types=[
            pltpu.VMEM((L * 16,), jnp.int32),
            pltpu.VMEM((256,), jnp.int32),
        ],
    )
    def sc_count(seq_hbm, out_hbm, seq_v, out_v):
        wid = lax.axis_index("s") * num_cores + lax.axis_index("c")
        pltpu.sync_copy(seq_hbm.at[pl.ds(wid * (L * 16), L * 16)], seq_v)

        iota = lax.iota(jnp.int32, 16)
        zero = jnp.zeros((16,), jnp.int32)
        one = jnp.ones((16,), jnp.int32)
        mcol = iota & 3
        mrow = iota >> 2

        def step(st, c):
            n1r, n2, n30, n31, n32, s2 = st
            s2 = s2 + n2
            n30 = n30 + jnp.where(c == 0, n2, zero)
            n31 = n31 + jnp.where(c == 1, n2, zero)
            n32 = n32 + jnp.where(c == 2, n2, zero)
            n2 = n2 + jnp.where(c == mcol, n1r, zero)
            n1r = n1r + jnp.where(c == mrow, one, zero)
            return (n1r, n2, n30, n31, n32, s2)

        def body(i, carry):
            stA, stB = carry[:6], carry[6:]
            base = i * 16 * 16
            for u in range(16):
                stA = step(stA, seq_v[pl.ds(base + u * 16, 16)])
                stB = step(stB, seq_v[pl.ds(base + u * 16 + H * 16, 16)])
            return stA + stB

        st = lax.fori_loop(0, H // 16, body, (zero,) * 12)

        for half in range(2):
            n1r, n2, n30, n31, n32, s2 = st[half * 6:half * 6 + 6]
            o = half * 128
            out_v[pl.ds(o + 0, 16)] = n1r
            out_v[pl.ds(o + 16, 16)] = n2
            out_v[pl.ds(o + 32, 16)] = n30
            out_v[pl.ds(o + 48, 16)] = n31
            out_v[pl.ds(o + 64, 16)] = n32
            out_v[pl.ds(o + 80, 16)] = s2 - n30 - n31 - n32
            out_v[pl.ds(o + 96, 16)] = zero
            out_v[pl.ds(o + 112, 16)] = zero
        pltpu.sync_copy(out_v, out_hbm.at[pl.ds(wid * 256, 256)])

    return sc_count, 2 * W


def _tc_finalize_body(W, n_total,
                      p_ref, h_ref, s_ref, gs_ref,
                      cs_ref, dw_ref, cp_ref, o_ref):
    f32 = jnp.float32
    dot = functools.partial(jnp.dot, preferred_element_type=f32,
                            precision=lax.Precision.HIGHEST)

    Pf = p_ref[...].astype(f32)
    i160 = lax.broadcasted_iota(jnp.int32, (16, 4), 0)
    i161 = lax.broadcasted_iota(jnp.int32, (16, 4), 1)
    SEL = (i160 == i161 * 4).astype(f32)
    N1b = dot(Pf[:, 0:16], SEL)
    N2b = Pf[:, 16:32]
    N3b = Pf[:, 32:96]

    iw0 = lax.broadcasted_iota(jnp.int32, (W, W), 0)
    iw1 = lax.broadcasted_iota(jnp.int32, (W, W), 1)
    Lstrict = (iw1 < iw0).astype(f32)

    N1pre = dot(Lstrict, N1b)

    i40 = lax.broadcasted_iota(jnp.int32, (4, 16), 0)
    i41 = lax.broadcasted_iota(jnp.int32, (4, 16), 1)
    E4 = (i41 // 4 == i40).astype(f32)
    F4 = (i41 % 4 == i40).astype(f32)
    N1pre16 = dot(N1pre, E4)
    N1b16 = dot(N1b, F4)
    S2b = N2b + N1pre16 * N1b16
    N2pre = dot(Lstrict, S2b)

    ia0 = lax.broadcasted_iota(jnp.int32, (4, 64), 0)
    ia1 = lax.broadcasted_iota(jnp.int32, (4, 64), 1)
    E16 = (ia1 // 16 == ia0).astype(f32)
    ig0 = lax.broadcasted_iota(jnp.int32, (16, 64), 0)
    ig1 = lax.broadcasted_iota(jnp.int32, (16, 64), 1)
    Etile = (ig1 % 16 == ig0).astype(f32)

    ones_w = jnp.ones((1, W), f32)
    A64 = dot(ones_w, N3b)
    Bflat = dot(ones_w, dot(N1b, E16) * dot(N2pre, Etile))
    ABflat = A64 + Bflat
    Cflat = dot(ones_w, dot(N1pre, E16) * dot(N2b, Etile))

    f0 = lax.broadcasted_iota(jnp.int32, (64, 64), 0)
    d1 = lax.broadcasted_iota(jnp.int32, (64, 64), 1)

    def cell_matrix(aa, bb, cc):
        sh = jnp.zeros((64, 64), jnp.int32)
        wv = jnp.ones((64, 64), f32)
        for k in range(4):
            sh = sh + jnp.where(aa == k, h_ref[0, k], 0)
            sh = sh + jnp.where(bb == k, h_ref[1, k], 0)
            sh = sh + jnp.where(cc == k, h_ref[2, k], 0)
            wv = wv * jnp.where(aa == k, s_ref[0, k], 1.0)
            wv = wv * jnp.where(bb == k, s_ref[1, k], 1.0)
            wv = wv * jnp.where(cc == k, s_ref[2, k], 1.0)
        return (sh % 64 == d1).astype(f32) * wv

    M_AB = cell_matrix((f0 // 4) % 4, f0 % 4, f0 // 16)
    M_C = cell_matrix(f0 // 16, (f0 // 4) % 4, f0 % 4)
    sk = dot(ABflat, M_AB) + dot(Cflat, M_C)

    N1tot = dot(ones_w, N1b)
    probs = N1tot / float(n_total)
    scaling = 0.9 + 0.2 * jnp.sum(probs * cs_ref[...])
    nrm = jnp.sqrt(jnp.sum(sk * sk))
    base = sk / (nrm + 1e-8)
    out = base * gs_ref[0, 0] * scaling * dw_ref[...]
    out = out + 0.1 * dot(probs, cp_ref[...])
    o_ref[...] = out


def _tc_finalize(W, n_total, p, h_i, s_f, gs, cs, dw, cp):
    smem = pl.BlockSpec(memory_space=pltpu.SMEM)
    vmem = pl.BlockSpec(memory_space=pltpu.VMEM)
    return pl.pallas_call(
        functools.partial(_tc_finalize_body, W, n_total),
        out_shape=jax.ShapeDtypeStruct((1, 64), jnp.float32),
        in_specs=[vmem, smem, smem, smem, vmem, vmem, vmem],
    )(p, h_i, s_f, gs, cs, dw, cp)


def kernel(sequence, h, s, global_scale, char_scales, dim_weights,
           char_perturbations):
    n_total = sequence.shape[0]
    info = plsc.get_sparse_core_info()
    sc_count, W = _make_sc_counter(n_total, info.num_cores, info.num_subcores)

    seq2d = sequence.astype(jnp.int32).reshape(n_total // 128, 128)
    seq_exp = _tc_expand(seq2d).reshape(n_total * 16)
    partials = sc_count(seq_exp).reshape(W, 128)

    out = _tc_finalize(
        W, n_total, partials,
        h.astype(jnp.int32), s.astype(jnp.float32),
        jnp.reshape(global_scale, (1, 1)).astype(jnp.float32),
        jnp.reshape(char_scales, (1, 4)).astype(jnp.float32),
        jnp.reshape(dim_weights, (1, 64)).astype(jnp.float32),
        char_perturbations.astype(jnp.float32),
    )
    return out.reshape(64)
